# baseline (speedup 1.0000x reference)
import functools

import jax
import jax.numpy as jnp
from jax import lax
from jax.experimental import pallas as pl
from jax.experimental.pallas import tpu as pltpu
from jax.experimental.pallas import tpu_sc as plsc

VOCAB = 256
D = 512
B = 4096 * 50

_info = plsc.get_sparse_core_info()
NC, NS = _info.num_cores, _info.num_subcores
NW = NC * NS
B_PER_W = B // NW
CHUNK = 80
NCHUNKS = B_PER_W // CHUNK


def _body(idx_hbm, table_hbm, out_hbm, idx_v, buf_v, sem0, sem1):
    wid = lax.axis_index("s") * NC + lax.axis_index("c")
    base = wid * B_PER_W
    pltpu.sync_copy(idx_hbm.at[pl.ds(base, B_PER_W)], idx_v)
    sems = [sem0, sem1]

    def gather_start(i, b):
        pltpu.make_async_copy(
            table_hbm.at[idx_v.at[pl.ds(i * CHUNK, CHUNK)]], buf_v.at[b], sems[b]
        ).start()

    def gather_wait(b):
        pltpu.make_async_copy(
            table_hbm.at[idx_v.at[pl.ds(0, CHUNK)]], buf_v.at[b], sems[b]
        ).wait()

    gather_start(0, 0)
    gather_start(1, 1)

    def pair_body(g, carry):
        for b in range(2):
            i = g * 2 + b
            gather_wait(b)
            pltpu.sync_copy(buf_v.at[b], out_hbm.at[pl.ds(base + i * CHUNK, CHUNK)])

            @pl.when(i + 2 < NCHUNKS)
            def _():
                gather_start(i + 2, b)

        return carry

    lax.fori_loop(0, NCHUNKS // 2, pair_body, 0)


@jax.jit
def _lookup(idx, table):
    mesh = plsc.VectorSubcoreMesh(core_axis_name="c", subcore_axis_name="s")
    run = functools.partial(
        pl.kernel,
        out_type=jax.ShapeDtypeStruct((B, D), jnp.float32),
        mesh=mesh,
        scratch_types=[
            pltpu.VMEM((B_PER_W,), jnp.int32),
            pltpu.VMEM((2, CHUNK, D), jnp.float32),
            pltpu.SemaphoreType.DMA,
            pltpu.SemaphoreType.DMA,
        ],
        compiler_params=pltpu.CompilerParams(use_tc_tiling_on_sc=True),
    )(_body)
    return run(idx, table)


def kernel(x, embedding):
    idx = x.reshape(-1).astype(jnp.int32)
    out = _lookup(idx, embedding)
    return out.reshape(x.shape + (embedding.shape[1],))


# speedup vs baseline: 1.9472x; 1.9472x over previous
import functools

import jax
import jax.numpy as jnp
from jax import lax
from jax.experimental import pallas as pl
from jax.experimental.pallas import tpu as pltpu

VOCAB = 256
D = 512
XR = 4096
S = 50
RB = 16


def _tc_body(x_ref, t_ref, o_ref):
    t = t_ref[...]
    for j in range(RB):
        row = x_ref[j]
        oh = (row[:, None] == lax.broadcasted_iota(jnp.int32, (S, VOCAB), 1)).astype(
            jnp.float32
        )
        o_ref[j] = jnp.dot(oh, t, preferred_element_type=jnp.float32)


@jax.jit
def _tc_lookup(x, table):
    return pl.pallas_call(
        _tc_body,
        grid=(XR // RB,),
        in_specs=[
            pl.BlockSpec((RB, S), lambda i: (i, 0)),
            pl.BlockSpec((VOCAB, D), lambda i: (0, 0)),
        ],
        out_specs=pl.BlockSpec((RB, S, D), lambda i: (i, 0, 0)),
        out_shape=jax.ShapeDtypeStruct((XR, S, D), jnp.float32),
    )(x, table)


def kernel(x, embedding):
    return _tc_lookup(x.astype(jnp.int32), embedding)
